# Initial kernel scaffold; baseline (speedup 1.0000x reference)
import jax
import jax.numpy as jnp
from jax.experimental import pallas as pl


def kernel(feat, edge_index, bag_indices, labels, W1, b1, W2, b2, W3, b3, ln_g, ln_b, Wh1, bh1, Wh2, bh2, Wp1, bp1, Wp2, bp2, Wc1, bc1, lnc_g, lnc_b, Wc2, bc2):
    raise NotImplementedError("write your pallas kernel here")


# speedup vs baseline: 1.1844x; 1.1844x over previous
import functools

import jax
import jax.numpy as jnp
from jax import lax
from jax.experimental import pallas as pl

N = 10000
E = 320000
IN_DIM = 128
HID = 256
OUT = 128
NB = 64
BS = 100
NC = 2
TEMP = 0.5

ROW_BLK = 400


def _gelu(x):
    return jax.nn.gelu(x, approximate=False)


def _ln(x, g, b):
    mu = x.mean(-1, keepdims=True)
    var = ((x - mu) ** 2).mean(-1, keepdims=True)
    return (x - mu) / jnp.sqrt(var + 1e-5) * g + b


def _normalize(x):
    n = jnp.linalg.norm(x, axis=-1, keepdims=True)
    return x / jnp.maximum(n, 1e-12)


def _contrast_kernel(z_blk_ref, z_all_ref, out_ref):
    i = pl.program_id(0)
    z_blk = z_blk_ref[...]
    z_all = z_all_ref[...]
    s = lax.dot_general(
        z_blk, z_all, (((1,), (1,)), ((), ())),
        preferred_element_type=jnp.float32,
        precision=lax.Precision.HIGHEST,
    ) * (1.0 / TEMP)
    m = jnp.max(s, axis=1, keepdims=True)
    lse = jnp.log(jnp.sum(jnp.exp(s - m), axis=1, keepdims=True)) + m
    diag = jnp.sum(z_blk * z_blk, axis=1, keepdims=True) * (1.0 / TEMP)
    part = jnp.sum(lse - diag, axis=(0, 1), keepdims=True)

    @pl.when(i == 0)
    def _():
        out_ref[...] = jnp.zeros_like(out_ref)

    out_ref[...] += part


def _contrast(z):
    nblk = N // ROW_BLK
    total = pl.pallas_call(
        _contrast_kernel,
        grid=(nblk,),
        in_specs=[
            pl.BlockSpec((ROW_BLK, OUT), lambda i: (i, 0)),
            pl.BlockSpec((N, OUT), lambda i: (0, 0)),
        ],
        out_specs=pl.BlockSpec((1, 1), lambda i: (0, 0)),
        out_shape=jax.ShapeDtypeStruct((1, 1), jnp.float32),
    )(z, z)
    return total[0, 0]


def kernel(feat, edge_index, bag_indices, labels, W1, b1, W2, b2, W3, b3,
           ln_g, ln_b, Wh1, bh1, Wh2, bh2, Wp1, bp1, Wp2, bp2, Wc1, bc1,
           lnc_g, lnc_b, Wc2, bc2):
    src = edge_index[0]
    dst = edge_index[1]
    ones = jnp.ones((E,), jnp.float32)
    deg_out = jnp.maximum(jax.ops.segment_sum(ones, src, num_segments=N), 1.0)
    deg_in = jnp.maximum(jax.ops.segment_sum(ones, dst, num_segments=N), 1.0)
    ns = deg_out ** -0.5
    nd = deg_in ** -0.5

    def agg(y):
        return jax.ops.segment_sum(y[src], dst, num_segments=N)

    h = _gelu((agg(feat * ns[:, None]) * nd[:, None]) @ W1 + b1)
    h = _gelu((agg(h * ns[:, None]) * nd[:, None]) @ W2 + b2)
    h = agg((h @ W3) * ns[:, None]) * nd[:, None] + b3
    h = _ln(h, ln_g, ln_b)

    z = _normalize(_gelu(h @ Wp1 + bp1) @ Wp2 + bp2)
    total = _contrast(z)
    contrast = total / N

    a = _gelu(jnp.einsum('nd,hdk->hnk', h, Wh1) + bh1[:, None, :])
    scores = (jnp.einsum('hnk,hko->hno', a, Wh2) + bh2[:, None, :]).mean(0)
    w = jax.nn.softmax(scores, axis=0)
    hb = h[bag_indices]
    wb = w[bag_indices]
    bag_feats = (wb * hb).sum(1)
    x = _gelu(_ln(bag_feats @ Wc1 + bc1, lnc_g, lnc_b))
    logits = x @ Wc2 + bc2
    lp = jax.nn.log_softmax(logits, axis=-1)
    nll = -lp[jnp.arange(NB), labels]
    cls = (0.9 * nll + 0.1 * (-lp.mean(-1))).mean()
    total_loss = contrast * 0.6 + cls * 0.4
    return logits, total_loss


# speedup vs baseline: 4.1089x; 3.4692x over previous
import functools

import jax
import jax.numpy as jnp
from jax import lax
from jax.experimental import pallas as pl
from jax.experimental.pallas import tpu as pltpu
from jax.experimental.pallas import tpu_sc as plsc

N = 10000
E = 320000
IN_DIM = 128
HID = 256
OUT = 128
NB = 64
BS = 100
NC = 2
TEMP = 0.5

NCORES = 2
NSUB = 16
EW = E // 128
ROW_BLK = 400
BLK = 2000

_HIGH = lax.Precision.HIGHEST


def _mesh():
    return plsc.VectorSubcoreMesh(core_axis_name="c", subcore_axis_name="s")


def _gelu(x):
    return 0.5 * x * (1.0 + lax.erf(x * (2.0 ** -0.5)))


def _layernorm(x, g, b):
    mu = x.mean(-1, keepdims=True)
    var = ((x - mu) ** 2).mean(-1, keepdims=True)
    return (x - mu) / jnp.sqrt(var + 1e-5) * g + b


def _dot(a, b):
    return lax.dot_general(a, b, (((1,), (0,)), ((), ())),
                           preferred_element_type=jnp.float32,
                           precision=_HIGH)


def _row_part(s, n_out, fn):
    if n_out // NSUB >= 8:
        ch = (n_out // NSUB) // 8 * 8
        fn(pl.multiple_of(s * ch, 8), ch)
        tail = n_out - ch * NSUB
        if tail:
            @pl.when(s == 0)
            def _():
                fn(ch * NSUB, tail)
    else:
        nse = n_out // 8

        @pl.when(s < nse)
        def _():
            fn(pl.multiple_of(s * 8, 8), 8)


def _widx(w):
    return pl.ds(pl.multiple_of(w * 128, 8), 128)


def _sc_hist(idx2, zeros, ones):
    ew = EW
    wpt = -(-ew // NSUB)

    @functools.partial(
        pl.kernel,
        out_type=jax.ShapeDtypeStruct((NCORES, N, 128), jnp.float32),
        mesh=_mesh(),
        scratch_types=[
            pltpu.VMEM((1, 128), jnp.int32),
            pltpu.VMEM((128, 128), jnp.float32),
            pltpu.VMEM_SHARED((N, 128), jnp.float32),
        ],
    )
    def k(idx_h, z_h, ones_h, out_h, idx, ones_v, hist):
        c = lax.axis_index("c")
        s = lax.axis_index("s")
        pltpu.sync_copy(ones_h, ones_v)
        _row_part(s, N, lambda st, sz: pltpu.sync_copy(
            z_h.at[pl.ds(st, sz)], hist.at[pl.ds(st, sz)]))
        plsc.subcore_barrier()
        base = s * wpt
        coff = c * (ew * 128)

        @pl.loop(0, wpt)
        def _(j):
            w = base + j
            wc = jnp.minimum(w, ew - 1)
            pltpu.sync_copy(
                idx_h.at[pl.ds(pl.multiple_of(coff + wc * 128, 8), 128)],
                idx.at[0])

            @pl.when(w < ew)
            def _():
                pltpu.sync_copy(ones_v, hist.at[idx.at[0]], add=True)

        plsc.subcore_barrier()
        _row_part(s, N, lambda st, sz: pltpu.sync_copy(
            hist.at[pl.ds(st, sz)], out_h.at[c, pl.ds(st, sz)]))

    return k(idx2, zeros, ones)


def _sc_segsum_split(y, src, dst, zeros):
    ew = src.shape[0] // 128
    n_out = zeros.shape[0]
    wpc = -(-ew // NCORES)
    wpt = -(-wpc // NSUB)

    @functools.partial(
        pl.kernel,
        out_type=jax.ShapeDtypeStruct((NCORES, n_out, 128), jnp.float32),
        mesh=_mesh(),
        scratch_types=[
            pltpu.VMEM((1, 128), jnp.int32),
            pltpu.VMEM((1, 128), jnp.int32),
            pltpu.VMEM((128, 128), jnp.float32),
            pltpu.VMEM_SHARED((n_out, 128), jnp.float32),
        ],
    )
    def k(y_h, src_h, dst_h, z_h, out_h, sidx, didx, rows, acc):
        c = lax.axis_index("c")
        s = lax.axis_index("s")
        _row_part(s, n_out, lambda st, sz: pltpu.sync_copy(
            z_h.at[pl.ds(st, sz)], acc.at[pl.ds(st, sz)]))
        plsc.subcore_barrier()
        lim = jnp.minimum((c + 1) * wpc, ew)
        base = c * wpc + s * wpt

        @pl.loop(0, wpt)
        def _(j):
            w = base + j
            wc = jnp.minimum(w, ew - 1)
            pltpu.sync_copy(src_h.at[_widx(wc)], sidx.at[0])
            pltpu.sync_copy(dst_h.at[_widx(wc)], didx.at[0])
            pltpu.sync_copy(y_h.at[sidx.at[0]], rows)

            @pl.when(w < lim)
            def _():
                pltpu.sync_copy(rows, acc.at[didx.at[0]], add=True)

        plsc.subcore_barrier()
        _row_part(s, n_out, lambda st, sz: pltpu.sync_copy(
            acc.at[pl.ds(st, sz)], out_h.at[c, pl.ds(st, sz)]))

    return k(y, src, dst, zeros)


def _sc_segsum_feat(ycat, src2, dst, zeros):
    ew = dst.shape[0] // 128
    n_out = zeros.shape[0]
    wpt = -(-ew // NSUB)

    @functools.partial(
        pl.kernel,
        out_type=jax.ShapeDtypeStruct((NCORES, n_out, 128), jnp.float32),
        mesh=_mesh(),
        scratch_types=[
            pltpu.VMEM((1, 128), jnp.int32),
            pltpu.VMEM((1, 128), jnp.int32),
            pltpu.VMEM((128, 128), jnp.float32),
            pltpu.VMEM_SHARED((n_out, 128), jnp.float32),
        ],
    )
    def k(ycat_h, src2_h, dst_h, z_h, out_h, sidx, didx, rows, acc):
        c = lax.axis_index("c")
        s = lax.axis_index("s")
        _row_part(s, n_out, lambda st, sz: pltpu.sync_copy(
            z_h.at[pl.ds(st, sz)], acc.at[pl.ds(st, sz)]))
        plsc.subcore_barrier()
        base = s * wpt
        coff = c * (ew * 128)

        @pl.loop(0, wpt)
        def _(j):
            w = base + j
            wc = jnp.minimum(w, ew - 1)
            pltpu.sync_copy(
                src2_h.at[pl.ds(pl.multiple_of(coff + wc * 128, 8), 128)],
                sidx.at[0])
            pltpu.sync_copy(dst_h.at[_widx(wc)], didx.at[0])
            pltpu.sync_copy(ycat_h.at[sidx.at[0]], rows)

            @pl.when(w < ew)
            def _():
                pltpu.sync_copy(rows, acc.at[didx.at[0]], add=True)

        plsc.subcore_barrier()
        _row_part(s, n_out, lambda st, sz: pltpu.sync_copy(
            acc.at[pl.ds(st, sz)], out_h.at[c, pl.ds(st, sz)]))

    return k(ycat, src2, dst, zeros)


def _ns_from(hist_blk):
    return lax.rsqrt(jnp.maximum(hist_blk[:, :1], 1.0))


def _prep_body(feat_r, hs_r, y_r):
    y_r[...] = feat_r[...] * _ns_from(hs_r[...])


def _prep(feat, hs):
    return pl.pallas_call(
        _prep_body,
        grid=(N // BLK,),
        in_specs=[
            pl.BlockSpec((BLK, 128), lambda i: (i, 0)),
            pl.BlockSpec((BLK, 128), lambda i: (i, 0)),
        ],
        out_specs=pl.BlockSpec((BLK, 128), lambda i: (i, 0)),
        out_shape=jax.ShapeDtypeStruct((N, 128), jnp.float32),
    )(feat, hs)


def _post1_body(a0_r, a1_r, hd_r, hs_r, W1_r, b1_r, y2_r):
    nd = _ns_from(hd_r[...])
    ns = _ns_from(hs_r[...])
    agg = (a0_r[...] + a1_r[...]) * nd
    h = _gelu(_dot(agg, W1_r[...]) + b1_r[...])
    y = h * ns
    y2_r[0] = y[:, :128]
    y2_r[1] = y[:, 128:]


def _post1(a0, a1, hd, hs, W1, b1):
    return pl.pallas_call(
        _post1_body,
        grid=(N // BLK,),
        in_specs=[
            pl.BlockSpec((BLK, 128), lambda i: (i, 0)),
            pl.BlockSpec((BLK, 128), lambda i: (i, 0)),
            pl.BlockSpec((BLK, 128), lambda i: (i, 0)),
            pl.BlockSpec((BLK, 128), lambda i: (i, 0)),
            pl.BlockSpec((128, HID), lambda i: (0, 0)),
            pl.BlockSpec((1, HID), lambda i: (0, 0)),
        ],
        out_specs=pl.BlockSpec((2, BLK, 128), lambda i: (0, i, 0)),
        out_shape=jax.ShapeDtypeStruct((2, N, 128), jnp.float32),
    )(a0, a1, hd, hs, W1, b1.reshape(1, HID))


def _post2_body(a0_r, a1_r, hd_r, hs_r, W2_r, b2_r, W3_r, t_r):
    nd = _ns_from(hd_r[...])
    ns = _ns_from(hs_r[...])
    W2 = W2_r[...]
    u = _gelu(_dot(a0_r[...] * nd, W2[:128, :]) +
              _dot(a1_r[...] * nd, W2[128:, :]) + b2_r[...])
    t_r[...] = _dot(u, W3_r[...]) * ns


def _post2(a0, a1, hd, hs, W2, b2, W3):
    return pl.pallas_call(
        _post2_body,
        grid=(N // BLK,),
        in_specs=[
            pl.BlockSpec((BLK, 128), lambda i: (i, 0)),
            pl.BlockSpec((BLK, 128), lambda i: (i, 0)),
            pl.BlockSpec((BLK, 128), lambda i: (i, 0)),
            pl.BlockSpec((BLK, 128), lambda i: (i, 0)),
            pl.BlockSpec((HID, HID), lambda i: (0, 0)),
            pl.BlockSpec((1, HID), lambda i: (0, 0)),
            pl.BlockSpec((HID, 128), lambda i: (0, 0)),
        ],
        out_specs=pl.BlockSpec((BLK, 128), lambda i: (i, 0)),
        out_shape=jax.ShapeDtypeStruct((N, 128), jnp.float32),
    )(a0, a1, hd, hs, W2, b2.reshape(1, HID), W3)


def _post3_body(a0_r, a1_r, hd_r, b3_r, lng_r, lnb_r, Wp1_r, bp1_r, Wp2_r,
                bp2_r, Wh1_r, bh1_r, wv_r, cb_r, h_r, z_r, sc_r):
    nd = _ns_from(hd_r[...])
    hh = _layernorm((a0_r[...] + a1_r[...]) * nd + b3_r[...],
                    lng_r[...], lnb_r[...])
    h_r[...] = hh
    zz = _dot(_gelu(_dot(hh, Wp1_r[...]) + bp1_r[...]), Wp2_r[...]) + bp2_r[...]
    nrm = jnp.sqrt(jnp.sum(zz * zz, axis=1, keepdims=True))
    z_r[...] = zz / jnp.maximum(nrm, 1e-12)
    A = _gelu(_dot(hh, Wh1_r[...]) + bh1_r[...])
    sc_r[...] = _dot(A, wv_r[...]) + cb_r[...]


def _post3(a0, a1, hd, b3, ln_g, ln_b, Wp1, bp1, Wp2, bp2, Wh1f, bh1f, wvp, cbv):
    row = pl.BlockSpec((BLK, 128), lambda i: (i, 0))
    c128 = pl.BlockSpec((1, 128), lambda i: (0, 0))
    return pl.pallas_call(
        _post3_body,
        grid=(N // BLK,),
        in_specs=[
            row, row,
            pl.BlockSpec((BLK, 128), lambda i: (i, 0)),
            c128, c128, c128,
            pl.BlockSpec((128, 128), lambda i: (0, 0)), c128,
            pl.BlockSpec((128, 128), lambda i: (0, 0)), c128,
            pl.BlockSpec((128, 512), lambda i: (0, 0)),
            pl.BlockSpec((1, 512), lambda i: (0, 0)),
            pl.BlockSpec((512, 128), lambda i: (0, 0)),
            c128,
        ],
        out_specs=[row, row, row],
        out_shape=[
            jax.ShapeDtypeStruct((N, 128), jnp.float32),
            jax.ShapeDtypeStruct((N, 128), jnp.float32),
            jax.ShapeDtypeStruct((N, 128), jnp.float32),
        ],
    )(a0, a1, hd, b3.reshape(1, 128), ln_g.reshape(1, 128),
      ln_b.reshape(1, 128), Wp1, bp1.reshape(1, 128), Wp2,
      bp2.reshape(1, 128), Wh1f, bh1f, wvp, cbv)


def _softmaxw_body(sc_r, h_r, w_r):
    s0 = sc_r[:, :1]
    m = jnp.max(s0)
    e = jnp.exp(s0 - m)
    w = e / jnp.sum(e)
    w_r[...] = w * h_r[...]


def _softmaxw(sc, h):
    return pl.pallas_call(
        _softmaxw_body,
        in_specs=[
            pl.BlockSpec((N, 128), lambda: (0, 0)),
            pl.BlockSpec((N, 128), lambda: (0, 0)),
        ],
        out_specs=pl.BlockSpec((N, 128), lambda: (0, 0)),
        out_shape=jax.ShapeDtypeStruct((N, 128), jnp.float32),
    )(sc, h)


def _contrast_kernel(z_blk_ref, z_all_ref, out_ref):
    i = pl.program_id(0)
    z_blk = z_blk_ref[...]
    s = lax.dot_general(
        z_blk, z_all_ref[...], (((1,), (1,)), ((), ())),
        preferred_element_type=jnp.float32,
        precision=_HIGH,
    ) * (1.0 / TEMP)
    m = jnp.max(s, axis=1, keepdims=True)
    lse = jnp.log(jnp.sum(jnp.exp(s - m), axis=1, keepdims=True)) + m
    diag = jnp.sum(z_blk * z_blk, axis=1, keepdims=True) * (1.0 / TEMP)
    part = jnp.sum(lse - diag, axis=(0, 1), keepdims=True)

    @pl.when(i == 0)
    def _():
        out_ref[...] = jnp.zeros_like(out_ref)

    out_ref[...] += part


def _contrast(z):
    return pl.pallas_call(
        _contrast_kernel,
        grid=(N // ROW_BLK,),
        in_specs=[
            pl.BlockSpec((ROW_BLK, OUT), lambda i: (i, 0)),
            pl.BlockSpec((N, OUT), lambda i: (0, 0)),
        ],
        out_specs=pl.BlockSpec((1, 1), lambda i: (0, 0)),
        out_shape=jax.ShapeDtypeStruct((1, 1), jnp.float32),
    )(z, z)


def _final_body(b0_r, b1_r, Wc1_r, bc1_r, lg_r, lb_r, Wc2_r, bc2_r, oh_r,
                tot_r, logits_r, loss_r):
    bf = b0_r[...] + b1_r[...]
    x = _gelu(_layernorm(_dot(bf, Wc1_r[...]) + bc1_r[...], lg_r[...], lb_r[...]))
    logits = _dot(x, Wc2_r[...]) + bc2_r[...]
    logits_r[...] = logits
    m = jnp.max(logits, axis=1, keepdims=True)
    lse = jnp.log(jnp.sum(jnp.exp(logits - m), axis=1, keepdims=True)) + m
    lp = logits - lse
    nll = -jnp.sum(lp * oh_r[...], axis=1, keepdims=True)
    pmean = -jnp.mean(lp, axis=1, keepdims=True)
    cls = jnp.sum(0.9 * nll + 0.1 * pmean, axis=(0, 1), keepdims=True) / NB
    loss_r[...] = tot_r[...] * (0.6 / N) + cls * 0.4


def _final(b0, b1, Wc1, bc1, lnc_g, lnc_b, Wc2, bc2, oh, total):
    full = lambda s: pl.BlockSpec(s, lambda: tuple(0 for _ in s))
    return pl.pallas_call(
        _final_body,
        in_specs=[
            full((NB, 128)), full((NB, 128)),
            full((128, 128)), full((1, 128)), full((1, 128)), full((1, 128)),
            full((128, NC)), full((1, NC)), full((NB, NC)), full((1, 1)),
        ],
        out_specs=[full((NB, NC)), full((1, 1))],
        out_shape=[
            jax.ShapeDtypeStruct((NB, NC), jnp.float32),
            jax.ShapeDtypeStruct((1, 1), jnp.float32),
        ],
    )(b0, b1, Wc1, bc1.reshape(1, 128), lnc_g.reshape(1, 128),
      lnc_b.reshape(1, 128), Wc2, bc2.reshape(1, NC), oh, total)


def kernel(feat, edge_index, bag_indices, labels, W1, b1, W2, b2, W3, b3,
           ln_g, ln_b, Wh1, bh1, Wh2, bh2, Wp1, bp1, Wp2, bp2, Wc1, bc1,
           lnc_g, lnc_b, Wc2, bc2):
    srcw = edge_index[0]
    dstw = edge_index[1]
    zN = jnp.zeros((N, 128), jnp.float32)
    zB = jnp.zeros((NB, 128), jnp.float32)
    o128 = jnp.ones((128, 128), jnp.float32)

    hist = _sc_hist(edge_index.reshape(2 * E), zN, o128)
    hs, hd = hist[0], hist[1]

    y0 = _prep(feat, hs)
    agg1 = _sc_segsum_split(y0, srcw, dstw, zN)
    y2 = _post1(agg1[0], agg1[1], hd, hs, W1, b1)
    src2 = jnp.concatenate([srcw, srcw + N])
    agg2 = _sc_segsum_feat(y2.reshape(2 * N, 128), src2, dstw, zN)
    t = _post2(agg2[0], agg2[1], hd, hs, W2, b2, W3)
    agg3 = _sc_segsum_split(t, srcw, dstw, zN)

    Wh1f = Wh1.transpose(1, 0, 2).reshape(128, 512)
    bh1f = bh1.reshape(1, 512)
    wvp = jnp.pad(Wh2.reshape(512, 1) / 4.0, ((0, 0), (0, 127)))
    cbv = jnp.full((1, 128), jnp.mean(bh2), jnp.float32)
    h, z, sc = _post3(agg3[0], agg3[1], hd, b3, ln_g, ln_b, Wp1, bp1, Wp2,
                      bp2, Wh1f, bh1f, wvp, cbv)

    total = _contrast(z)
    weighted = _softmaxw(sc, h)

    bagw = bag_indices.reshape(NB * BS)
    bagidw = jnp.repeat(jnp.arange(NB, dtype=jnp.int32), BS)
    bagf = _sc_segsum_split(weighted, bagw, bagidw, zB)

    oh = (labels[:, None] == jnp.arange(NC, dtype=labels.dtype)[None, :]
          ).astype(jnp.float32)
    logits, loss = _final(bagf[0], bagf[1], Wc1, bc1, lnc_g, lnc_b, Wc2, bc2,
                          oh, total)
    return logits, loss[0, 0]


# speedup vs baseline: 5.5571x; 1.3524x over previous
import functools

import jax
import jax.numpy as jnp
from jax import lax
from jax.experimental import pallas as pl
from jax.experimental.pallas import tpu as pltpu
from jax.experimental.pallas import tpu_sc as plsc

N = 10000
E = 320000
IN_DIM = 128
HID = 256
OUT = 128
NB = 64
BS = 100
NC = 2
TEMP = 0.5

NCORES = 2
NSUB = 16
EW = E // 128
ROW_BLK = 400
BLK = 2000

_HIGH = lax.Precision.HIGHEST


def _mesh():
    return plsc.VectorSubcoreMesh(core_axis_name="c", subcore_axis_name="s")


def _gelu(x):
    return 0.5 * x * (1.0 + lax.erf(x * (2.0 ** -0.5)))


def _layernorm(x, g, b):
    mu = x.mean(-1, keepdims=True)
    var = ((x - mu) ** 2).mean(-1, keepdims=True)
    return (x - mu) / jnp.sqrt(var + 1e-5) * g + b


def _dot(a, b):
    return lax.dot_general(a, b, (((1,), (0,)), ((), ())),
                           preferred_element_type=jnp.float32,
                           precision=_HIGH)


def _row_part(s, n_out, fn):
    if n_out // NSUB >= 8:
        ch = (n_out // NSUB) // 8 * 8
        fn(pl.multiple_of(s * ch, 8), ch)
        tail = n_out - ch * NSUB
        if tail:
            @pl.when(s == 0)
            def _():
                fn(ch * NSUB, tail)
    else:
        nse = n_out // 8

        @pl.when(s < nse)
        def _():
            fn(pl.multiple_of(s * 8, 8), 8)


def _widx(w):
    return pl.ds(pl.multiple_of(w * 128, 8), 128)


SLAB = 8


def _slab_engine(s_rows, acc, sidx, didx, rows0, rows1, sems, wbase, lim,
                 gather_src):
    bufs = (rows0, rows1)
    descs = [None, None]
    for j in range(SLAB):
        if gather_src is not None:
            descs[j % 2] = pltpu.async_copy(
                gather_src(j), bufs[j % 2], sems[j % 2])
        if j > 0:
            jj = j - 1
            if gather_src is not None:
                descs[jj % 2].wait()

            @pl.when(wbase + jj < lim)
            def _(jj=jj):
                pltpu.sync_copy(bufs[jj % 2] if gather_src is not None
                                else rows0,
                                acc.at[didx.at[jj]], add=True)
    jj = SLAB - 1
    if gather_src is not None:
        descs[jj % 2].wait()

    @pl.when(wbase + jj < lim)
    def _():
        pltpu.sync_copy(bufs[jj % 2] if gather_src is not None else rows0,
                        acc.at[didx.at[jj]], add=True)


def _sc_hist(idx2w, zeros, ones, ew):
    pw = idx2w.shape[0] // 2
    wpt = -(-(-(-ew // NSUB)) // SLAB) * SLAB

    @functools.partial(
        pl.kernel,
        out_type=jax.ShapeDtypeStruct((NCORES, N, 128), jnp.float32),
        mesh=_mesh(),
        scratch_types=[
            pltpu.VMEM((SLAB, 128), jnp.int32),
            pltpu.VMEM((128, 128), jnp.float32),
            pltpu.VMEM_SHARED((N, 128), jnp.float32),
        ],
    )
    def k(idx_h, z_h, ones_h, out_h, didx, ones_v, hist):
        c = lax.axis_index("c")
        s = lax.axis_index("s")
        pltpu.sync_copy(ones_h, ones_v)
        _row_part(s, N, lambda st, sz: pltpu.sync_copy(
            z_h.at[pl.ds(st, sz)], hist.at[pl.ds(st, sz)]))
        plsc.subcore_barrier()
        base = s * wpt
        crow = c * pw

        @pl.loop(0, wpt // SLAB)
        def _(kk):
            wbase = base + kk * SLAB

            @pl.when(wbase < ew)
            def _():
                pltpu.sync_copy(
                    idx_h.at[pl.ds(pl.multiple_of(crow + wbase, 8), SLAB)],
                    didx)
                _slab_engine(None, hist, None, didx, ones_v, None, None,
                             wbase, ew, None)

        plsc.subcore_barrier()
        _row_part(s, N, lambda st, sz: pltpu.sync_copy(
            hist.at[pl.ds(st, sz)], out_h.at[c, pl.ds(st, sz)]))

    return k(idx2w, zeros, ones)


def _sc_segsum_split(y, srcw, dstw, zeros, ew):
    n_out = zeros.shape[0]
    wpc = -(-(-(-ew // NCORES)) // SLAB) * SLAB
    wpt = -(-(-(-wpc // NSUB)) // SLAB) * SLAB

    @functools.partial(
        pl.kernel,
        out_type=jax.ShapeDtypeStruct((NCORES, n_out, 128), jnp.float32),
        mesh=_mesh(),
        scratch_types=[
            pltpu.VMEM((SLAB, 128), jnp.int32),
            pltpu.VMEM((SLAB, 128), jnp.int32),
            pltpu.VMEM((128, 128), jnp.float32),
            pltpu.VMEM((128, 128), jnp.float32),
            pltpu.SemaphoreType.DMA,
            pltpu.SemaphoreType.DMA,
            pltpu.VMEM_SHARED((n_out, 128), jnp.float32),
        ],
    )
    def k(y_h, src_h, dst_h, z_h, out_h, sidx, didx, rows0, rows1, sem0,
          sem1, acc):
        c = lax.axis_index("c")
        s = lax.axis_index("s")
        _row_part(s, n_out, lambda st, sz: pltpu.sync_copy(
            z_h.at[pl.ds(st, sz)], acc.at[pl.ds(st, sz)]))
        plsc.subcore_barrier()
        lim = jnp.minimum((c + 1) * wpc, ew)
        base = c * wpc + s * wpt

        @pl.loop(0, wpt // SLAB)
        def _(kk):
            wbase = base + kk * SLAB

            @pl.when(wbase < lim)
            def _():
                wb = pl.multiple_of(wbase, 8)
                pltpu.sync_copy(src_h.at[pl.ds(wb, SLAB)], sidx)
                pltpu.sync_copy(dst_h.at[pl.ds(wb, SLAB)], didx)
                _slab_engine(None, acc, sidx, didx, rows0, rows1,
                             (sem0, sem1), wbase, lim,
                             lambda j: y_h.at[sidx.at[j]])

        plsc.subcore_barrier()
        _row_part(s, n_out, lambda st, sz: pltpu.sync_copy(
            acc.at[pl.ds(st, sz)], out_h.at[c, pl.ds(st, sz)]))

    return k(y, srcw, dstw, zeros)


def _sc_segsum_feat(ycat, src2w, dstw, zeros, ew):
    pw = src2w.shape[0] // 2
    n_out = zeros.shape[0]
    wpt = -(-(-(-ew // NSUB)) // SLAB) * SLAB

    @functools.partial(
        pl.kernel,
        out_type=jax.ShapeDtypeStruct((NCORES, n_out, 128), jnp.float32),
        mesh=_mesh(),
        scratch_types=[
            pltpu.VMEM((SLAB, 128), jnp.int32),
            pltpu.VMEM((SLAB, 128), jnp.int32),
            pltpu.VMEM((128, 128), jnp.float32),
            pltpu.VMEM((128, 128), jnp.float32),
            pltpu.SemaphoreType.DMA,
            pltpu.SemaphoreType.DMA,
            pltpu.VMEM_SHARED((n_out, 128), jnp.float32),
        ],
    )
    def k(ycat_h, src_h, dst_h, z_h, out_h, sidx, didx, rows0, rows1, sem0,
          sem1, acc):
        c = lax.axis_index("c")
        s = lax.axis_index("s")
        _row_part(s, n_out, lambda st, sz: pltpu.sync_copy(
            z_h.at[pl.ds(st, sz)], acc.at[pl.ds(st, sz)]))
        plsc.subcore_barrier()
        base = s * wpt
        crow = c * pw

        @pl.loop(0, wpt // SLAB)
        def _(kk):
            wbase = base + kk * SLAB

            @pl.when(wbase < ew)
            def _():
                pltpu.sync_copy(
                    src_h.at[pl.ds(pl.multiple_of(crow + wbase, 8), SLAB)],
                    sidx)
                pltpu.sync_copy(
                    dst_h.at[pl.ds(pl.multiple_of(wbase, 8), SLAB)], didx)
                _slab_engine(None, acc, sidx, didx, rows0, rows1,
                             (sem0, sem1), wbase, ew,
                             lambda j: ycat_h.at[sidx.at[j]])

        plsc.subcore_barrier()
        _row_part(s, n_out, lambda st, sz: pltpu.sync_copy(
            acc.at[pl.ds(st, sz)], out_h.at[c, pl.ds(st, sz)]))

    return k(ycat, src2w, dstw, zeros)


def _ns_from(hist_blk):
    return lax.rsqrt(jnp.maximum(hist_blk[:, :1], 1.0))


def _prep_body(feat_r, hs_r, y_r):
    y_r[...] = feat_r[...] * _ns_from(hs_r[...])


def _prep(feat, hs):
    return pl.pallas_call(
        _prep_body,
        grid=(N // BLK,),
        in_specs=[
            pl.BlockSpec((BLK, 128), lambda i: (i, 0)),
            pl.BlockSpec((BLK, 128), lambda i: (i, 0)),
        ],
        out_specs=pl.BlockSpec((BLK, 128), lambda i: (i, 0)),
        out_shape=jax.ShapeDtypeStruct((N, 128), jnp.float32),
    )(feat, hs)


def _post1_body(a0_r, a1_r, hd_r, hs_r, W1_r, b1_r, y2_r):
    nd = _ns_from(hd_r[...])
    ns = _ns_from(hs_r[...])
    agg = (a0_r[...] + a1_r[...]) * nd
    h = _gelu(_dot(agg, W1_r[...]) + b1_r[...])
    y = h * ns
    y2_r[0] = y[:, :128]
    y2_r[1] = y[:, 128:]


def _post1(a0, a1, hd, hs, W1, b1):
    return pl.pallas_call(
        _post1_body,
        grid=(N // BLK,),
        in_specs=[
            pl.BlockSpec((BLK, 128), lambda i: (i, 0)),
            pl.BlockSpec((BLK, 128), lambda i: (i, 0)),
            pl.BlockSpec((BLK, 128), lambda i: (i, 0)),
            pl.BlockSpec((BLK, 128), lambda i: (i, 0)),
            pl.BlockSpec((128, HID), lambda i: (0, 0)),
            pl.BlockSpec((1, HID), lambda i: (0, 0)),
        ],
        out_specs=pl.BlockSpec((2, BLK, 128), lambda i: (0, i, 0)),
        out_shape=jax.ShapeDtypeStruct((2, N, 128), jnp.float32),
    )(a0, a1, hd, hs, W1, b1.reshape(1, HID))


def _post2_body(a0_r, a1_r, hd_r, hs_r, W2_r, b2_r, W3_r, t_r):
    nd = _ns_from(hd_r[...])
    ns = _ns_from(hs_r[...])
    W2 = W2_r[...]
    u = _gelu(_dot(a0_r[...] * nd, W2[:128, :]) +
              _dot(a1_r[...] * nd, W2[128:, :]) + b2_r[...])
    t_r[...] = _dot(u, W3_r[...]) * ns


def _post2(a0, a1, hd, hs, W2, b2, W3):
    return pl.pallas_call(
        _post2_body,
        grid=(N // BLK,),
        in_specs=[
            pl.BlockSpec((BLK, 128), lambda i: (i, 0)),
            pl.BlockSpec((BLK, 128), lambda i: (i, 0)),
            pl.BlockSpec((BLK, 128), lambda i: (i, 0)),
            pl.BlockSpec((BLK, 128), lambda i: (i, 0)),
            pl.BlockSpec((HID, HID), lambda i: (0, 0)),
            pl.BlockSpec((1, HID), lambda i: (0, 0)),
            pl.BlockSpec((HID, 128), lambda i: (0, 0)),
        ],
        out_specs=pl.BlockSpec((BLK, 128), lambda i: (i, 0)),
        out_shape=jax.ShapeDtypeStruct((N, 128), jnp.float32),
    )(a0, a1, hd, hs, W2, b2.reshape(1, HID), W3)


def _post3_body(a0_r, a1_r, hd_r, b3_r, lng_r, lnb_r, Wp1_r, bp1_r, Wp2_r,
                bp2_r, Wh1_r, bh1_r, wv_r, cb_r, h_r, z_r, sc_r):
    nd = _ns_from(hd_r[...])
    hh = _layernorm((a0_r[...] + a1_r[...]) * nd + b3_r[...],
                    lng_r[...], lnb_r[...])
    h_r[...] = hh
    zz = _dot(_gelu(_dot(hh, Wp1_r[...]) + bp1_r[...]), Wp2_r[...]) + bp2_r[...]
    nrm = jnp.sqrt(jnp.sum(zz * zz, axis=1, keepdims=True))
    z_r[...] = zz / jnp.maximum(nrm, 1e-12)
    A = _gelu(_dot(hh, Wh1_r[...]) + bh1_r[...])
    sc_r[...] = _dot(A, wv_r[...]) + cb_r[...]


def _post3(a0, a1, hd, b3, ln_g, ln_b, Wp1, bp1, Wp2, bp2, Wh1f, bh1f, wvp, cbv):
    row = pl.BlockSpec((BLK, 128), lambda i: (i, 0))
    c128 = pl.BlockSpec((1, 128), lambda i: (0, 0))
    return pl.pallas_call(
        _post3_body,
        grid=(N // BLK,),
        in_specs=[
            row, row,
            pl.BlockSpec((BLK, 128), lambda i: (i, 0)),
            c128, c128, c128,
            pl.BlockSpec((128, 128), lambda i: (0, 0)), c128,
            pl.BlockSpec((128, 128), lambda i: (0, 0)), c128,
            pl.BlockSpec((128, 512), lambda i: (0, 0)),
            pl.BlockSpec((1, 512), lambda i: (0, 0)),
            pl.BlockSpec((512, 128), lambda i: (0, 0)),
            c128,
        ],
        out_specs=[row, row, row],
        out_shape=[
            jax.ShapeDtypeStruct((N, 128), jnp.float32),
            jax.ShapeDtypeStruct((N, 128), jnp.float32),
            jax.ShapeDtypeStruct((N, 128), jnp.float32),
        ],
    )(a0, a1, hd, b3.reshape(1, 128), ln_g.reshape(1, 128),
      ln_b.reshape(1, 128), Wp1, bp1.reshape(1, 128), Wp2,
      bp2.reshape(1, 128), Wh1f, bh1f, wvp, cbv)


def _softmaxw_body(sc_r, h_r, w_r):
    s0 = sc_r[:, :1]
    m = jnp.max(s0)
    e = jnp.exp(s0 - m)
    w = e / jnp.sum(e)
    w_r[...] = w * h_r[...]


def _softmaxw(sc, h):
    return pl.pallas_call(
        _softmaxw_body,
        in_specs=[
            pl.BlockSpec((N, 128), lambda: (0, 0)),
            pl.BlockSpec((N, 128), lambda: (0, 0)),
        ],
        out_specs=pl.BlockSpec((N, 128), lambda: (0, 0)),
        out_shape=jax.ShapeDtypeStruct((N, 128), jnp.float32),
    )(sc, h)


def _contrast_kernel(z_blk_ref, z_all_ref, out_ref):
    i = pl.program_id(0)
    z_blk = z_blk_ref[...]
    s = lax.dot_general(
        z_blk, z_all_ref[...], (((1,), (1,)), ((), ())),
        preferred_element_type=jnp.float32,
        precision=_HIGH,
    ) * (1.0 / TEMP)
    m = jnp.max(s, axis=1, keepdims=True)
    lse = jnp.log(jnp.sum(jnp.exp(s - m), axis=1, keepdims=True)) + m
    diag = jnp.sum(z_blk * z_blk, axis=1, keepdims=True) * (1.0 / TEMP)
    part = jnp.sum(lse - diag, axis=(0, 1), keepdims=True)

    @pl.when(i == 0)
    def _():
        out_ref[...] = jnp.zeros_like(out_ref)

    out_ref[...] += part


def _contrast(z):
    return pl.pallas_call(
        _contrast_kernel,
        grid=(N // ROW_BLK,),
        in_specs=[
            pl.BlockSpec((ROW_BLK, OUT), lambda i: (i, 0)),
            pl.BlockSpec((N, OUT), lambda i: (0, 0)),
        ],
        out_specs=pl.BlockSpec((1, 1), lambda i: (0, 0)),
        out_shape=jax.ShapeDtypeStruct((1, 1), jnp.float32),
    )(z, z)


def _final_body(b0_r, b1_r, Wc1_r, bc1_r, lg_r, lb_r, Wc2_r, bc2_r, oh_r,
                tot_r, logits_r, loss_r):
    bf = b0_r[...] + b1_r[...]
    x = _gelu(_layernorm(_dot(bf, Wc1_r[...]) + bc1_r[...], lg_r[...], lb_r[...]))
    logits = _dot(x, Wc2_r[...]) + bc2_r[...]
    logits_r[...] = logits
    m = jnp.max(logits, axis=1, keepdims=True)
    lse = jnp.log(jnp.sum(jnp.exp(logits - m), axis=1, keepdims=True)) + m
    lp = logits - lse
    nll = -jnp.sum(lp * oh_r[...], axis=1, keepdims=True)
    pmean = -jnp.mean(lp, axis=1, keepdims=True)
    cls = jnp.sum(0.9 * nll + 0.1 * pmean, axis=(0, 1), keepdims=True) / NB
    loss_r[...] = tot_r[...] * (0.6 / N) + cls * 0.4


def _final(b0, b1, Wc1, bc1, lnc_g, lnc_b, Wc2, bc2, oh, total):
    full = lambda s: pl.BlockSpec(s, lambda: tuple(0 for _ in s))
    return pl.pallas_call(
        _final_body,
        in_specs=[
            full((NB, 128)), full((NB, 128)),
            full((128, 128)), full((1, 128)), full((1, 128)), full((1, 128)),
            full((128, NC)), full((1, NC)), full((NB, NC)), full((1, 1)),
        ],
        out_specs=[full((NB, NC)), full((1, 1))],
        out_shape=[
            jax.ShapeDtypeStruct((NB, NC), jnp.float32),
            jax.ShapeDtypeStruct((1, 1), jnp.float32),
        ],
    )(b0, b1, Wc1, bc1.reshape(1, 128), lnc_g.reshape(1, 128),
      lnc_b.reshape(1, 128), Wc2, bc2.reshape(1, NC), oh, total)


def kernel(feat, edge_index, bag_indices, labels, W1, b1, W2, b2, W3, b3,
           ln_g, ln_b, Wh1, bh1, Wh2, bh2, Wp1, bp1, Wp2, bp2, Wc1, bc1,
           lnc_g, lnc_b, Wc2, bc2):
    PW = 2560
    srcw2 = jnp.pad(edge_index[0], (0, PW * 128 - E)).reshape(PW, 128)
    dstw2 = jnp.pad(edge_index[1], (0, PW * 128 - E)).reshape(PW, 128)
    zN = jnp.zeros((N, 128), jnp.float32)
    zB = jnp.zeros((NB, 128), jnp.float32)
    o128 = jnp.ones((128, 128), jnp.float32)

    hist = _sc_hist(jnp.concatenate([srcw2, dstw2]), zN, o128, EW)
    hs, hd = hist[0], hist[1]

    y0 = _prep(feat, hs)
    agg1 = _sc_segsum_split(y0, srcw2, dstw2, zN, EW)
    y2 = _post1(agg1[0], agg1[1], hd, hs, W1, b1)
    src2w = jnp.concatenate([srcw2, srcw2 + N])
    agg2 = _sc_segsum_feat(y2.reshape(2 * N, 128), src2w, dstw2, zN, EW)
    t = _post2(agg2[0], agg2[1], hd, hs, W2, b2, W3)
    agg3 = _sc_segsum_split(t, srcw2, dstw2, zN, EW)

    Wh1f = Wh1.transpose(1, 0, 2).reshape(128, 512)
    bh1f = bh1.reshape(1, 512)
    wvp = jnp.pad(Wh2.reshape(512, 1) / 4.0, ((0, 0), (0, 127)))
    cbv = jnp.full((1, 128), jnp.mean(bh2), jnp.float32)
    h, z, sc = _post3(agg3[0], agg3[1], hd, b3, ln_g, ln_b, Wp1, bp1, Wp2,
                      bp2, Wh1f, bh1f, wvp, cbv)

    total = _contrast(z)
    weighted = _softmaxw(sc, h)

    PWB = 64
    bagw2 = jnp.pad(bag_indices.reshape(NB * BS),
                    (0, PWB * 128 - NB * BS)).reshape(PWB, 128)
    bagidw2 = jnp.pad(jnp.repeat(jnp.arange(NB, dtype=jnp.int32), BS),
                      (0, PWB * 128 - NB * BS)).reshape(PWB, 128)
    bagf = _sc_segsum_split(weighted, bagw2, bagidw2, zB, NB * BS // 128)

    oh = (labels[:, None] == jnp.arange(NC, dtype=labels.dtype)[None, :]
          ).astype(jnp.float32)
    logits, loss = _final(bagf[0], bagf[1], Wc1, bc1, lnc_g, lnc_b, Wc2, bc2,
                          oh, total)
    return logits, loss[0, 0]


# speedup vs baseline: 7.5909x; 1.3660x over previous
import functools

import jax
import jax.numpy as jnp
from jax import lax
from jax.experimental import pallas as pl
from jax.experimental.pallas import tpu as pltpu
from jax.experimental.pallas import tpu_sc as plsc

N = 10000
E = 320000
IN_DIM = 128
HID = 256
OUT = 128
NB = 64
BS = 100
NC = 2
TEMP = 0.5

NCORES = 2
NSUB = 16
EW = E // 128
ROW_BLK = 400
BLK = 2000

_PREC = lax.Precision.DEFAULT


def _mesh():
    return plsc.VectorSubcoreMesh(core_axis_name="c", subcore_axis_name="s")


def _gelu(x):
    return 0.5 * x * (1.0 + lax.erf(x * (2.0 ** -0.5)))


def _layernorm(x, g, b):
    mu = x.mean(-1, keepdims=True)
    var = ((x - mu) ** 2).mean(-1, keepdims=True)
    return (x - mu) / jnp.sqrt(var + 1e-5) * g + b


def _dot(a, b):
    return lax.dot_general(a, b, (((1,), (0,)), ((), ())),
                           preferred_element_type=jnp.float32,
                           precision=_PREC)


def _row_part(s, n_out, fn):
    if n_out // NSUB >= 8:
        ch = (n_out // NSUB) // 8 * 8
        fn(pl.multiple_of(s * ch, 8), ch)
        tail = n_out - ch * NSUB
        if tail:
            @pl.when(s == 0)
            def _():
                fn(ch * NSUB, tail)
    else:
        nse = n_out // 8

        @pl.when(s < nse)
        def _():
            fn(pl.multiple_of(s * 8, 8), 8)


def _widx(w):
    return pl.ds(pl.multiple_of(w * 128, 8), 128)


SLAB = 8


def _slab_engine(s_rows, acc, sidx, didx, rows0, rows1, sems, wbase, lim,
                 gather_src):
    bufs = (rows0, rows1)
    descs = [None, None]
    for j in range(SLAB):
        if gather_src is not None:
            descs[j % 2] = pltpu.async_copy(
                gather_src(j), bufs[j % 2], sems[j % 2])
        if j > 0:
            jj = j - 1
            if gather_src is not None:
                descs[jj % 2].wait()

            @pl.when(wbase + jj < lim)
            def _(jj=jj):
                pltpu.sync_copy(bufs[jj % 2] if gather_src is not None
                                else rows0,
                                acc.at[didx.at[jj]], add=True)
    jj = SLAB - 1
    if gather_src is not None:
        descs[jj % 2].wait()

    @pl.when(wbase + jj < lim)
    def _():
        pltpu.sync_copy(bufs[jj % 2] if gather_src is not None else rows0,
                        acc.at[didx.at[jj]], add=True)


def _sc_hist(idx2w, zeros, ones, ew):
    pw = idx2w.shape[0] // 2
    wpt = -(-(-(-ew // NSUB)) // SLAB) * SLAB

    @functools.partial(
        pl.kernel,
        out_type=jax.ShapeDtypeStruct((NCORES, N, 128), jnp.float32),
        mesh=_mesh(),
        scratch_types=[
            pltpu.VMEM((SLAB, 128), jnp.int32),
            pltpu.VMEM((128, 128), jnp.float32),
            pltpu.VMEM_SHARED((N, 128), jnp.float32),
        ],
    )
    def k(idx_h, z_h, ones_h, out_h, didx, ones_v, hist):
        c = lax.axis_index("c")
        s = lax.axis_index("s")
        pltpu.sync_copy(ones_h, ones_v)
        _row_part(s, N, lambda st, sz: pltpu.sync_copy(
            z_h.at[pl.ds(st, sz)], hist.at[pl.ds(st, sz)]))
        plsc.subcore_barrier()
        base = s * wpt
        crow = c * pw

        @pl.loop(0, wpt // SLAB)
        def _(kk):
            wbase = base + kk * SLAB

            @pl.when(wbase < ew)
            def _():
                pltpu.sync_copy(
                    idx_h.at[pl.ds(pl.multiple_of(crow + wbase, 8), SLAB)],
                    didx)
                _slab_engine(None, hist, None, didx, ones_v, None, None,
                             wbase, ew, None)

        plsc.subcore_barrier()
        _row_part(s, N, lambda st, sz: pltpu.sync_copy(
            hist.at[pl.ds(st, sz)], out_h.at[c, pl.ds(st, sz)]))

    return k(idx2w, zeros, ones)


def _sc_segsum_split(y, srcw, dstw, zeros, ew):
    n_out = zeros.shape[0]
    wpc = -(-(-(-ew // NCORES)) // SLAB) * SLAB
    wpt = -(-(-(-wpc // NSUB)) // SLAB) * SLAB

    @functools.partial(
        pl.kernel,
        out_type=jax.ShapeDtypeStruct((NCORES, n_out, 128), jnp.float32),
        mesh=_mesh(),
        scratch_types=[
            pltpu.VMEM((SLAB, 128), jnp.int32),
            pltpu.VMEM((SLAB, 128), jnp.int32),
            pltpu.VMEM((128, 128), jnp.float32),
            pltpu.VMEM((128, 128), jnp.float32),
            pltpu.SemaphoreType.DMA,
            pltpu.SemaphoreType.DMA,
            pltpu.VMEM_SHARED((n_out, 128), jnp.float32),
        ],
    )
    def k(y_h, src_h, dst_h, z_h, out_h, sidx, didx, rows0, rows1, sem0,
          sem1, acc):
        c = lax.axis_index("c")
        s = lax.axis_index("s")
        _row_part(s, n_out, lambda st, sz: pltpu.sync_copy(
            z_h.at[pl.ds(st, sz)], acc.at[pl.ds(st, sz)]))
        plsc.subcore_barrier()
        lim = jnp.minimum((c + 1) * wpc, ew)
        base = c * wpc + s * wpt

        @pl.loop(0, wpt // SLAB)
        def _(kk):
            wbase = base + kk * SLAB

            @pl.when(wbase < lim)
            def _():
                wb = pl.multiple_of(wbase, 8)
                pltpu.sync_copy(src_h.at[pl.ds(wb, SLAB)], sidx)
                pltpu.sync_copy(dst_h.at[pl.ds(wb, SLAB)], didx)
                _slab_engine(None, acc, sidx, didx, rows0, rows1,
                             (sem0, sem1), wbase, lim,
                             lambda j: y_h.at[sidx.at[j]])

        plsc.subcore_barrier()
        _row_part(s, n_out, lambda st, sz: pltpu.sync_copy(
            acc.at[pl.ds(st, sz)], out_h.at[c, pl.ds(st, sz)]))

    return k(y, srcw, dstw, zeros)


def _sc_segsum_feat(ycat, src2w, dstw, zeros, ew):
    pw = src2w.shape[0] // 2
    n_out = zeros.shape[0]
    wpt = -(-(-(-ew // NSUB)) // SLAB) * SLAB

    @functools.partial(
        pl.kernel,
        out_type=jax.ShapeDtypeStruct((NCORES, n_out, 128), jnp.float32),
        mesh=_mesh(),
        scratch_types=[
            pltpu.VMEM((SLAB, 128), jnp.int32),
            pltpu.VMEM((SLAB, 128), jnp.int32),
            pltpu.VMEM((128, 128), jnp.float32),
            pltpu.VMEM((128, 128), jnp.float32),
            pltpu.SemaphoreType.DMA,
            pltpu.SemaphoreType.DMA,
            pltpu.VMEM_SHARED((n_out, 128), jnp.float32),
        ],
    )
    def k(ycat_h, src_h, dst_h, z_h, out_h, sidx, didx, rows0, rows1, sem0,
          sem1, acc):
        c = lax.axis_index("c")
        s = lax.axis_index("s")
        _row_part(s, n_out, lambda st, sz: pltpu.sync_copy(
            z_h.at[pl.ds(st, sz)], acc.at[pl.ds(st, sz)]))
        plsc.subcore_barrier()
        base = s * wpt
        crow = c * pw

        @pl.loop(0, wpt // SLAB)
        def _(kk):
            wbase = base + kk * SLAB

            @pl.when(wbase < ew)
            def _():
                pltpu.sync_copy(
                    src_h.at[pl.ds(pl.multiple_of(crow + wbase, 8), SLAB)],
                    sidx)
                pltpu.sync_copy(
                    dst_h.at[pl.ds(pl.multiple_of(wbase, 8), SLAB)], didx)
                _slab_engine(None, acc, sidx, didx, rows0, rows1,
                             (sem0, sem1), wbase, ew,
                             lambda j: ycat_h.at[sidx.at[j]])

        plsc.subcore_barrier()
        _row_part(s, n_out, lambda st, sz: pltpu.sync_copy(
            acc.at[pl.ds(st, sz)], out_h.at[c, pl.ds(st, sz)]))

    return k(ycat, src2w, dstw, zeros)


def _ns_from(hist_blk):
    return lax.rsqrt(jnp.maximum(hist_blk[:, :1], 1.0))


def _prep_body(feat_r, hs_r, y_r):
    y_r[...] = feat_r[...] * _ns_from(hs_r[...])


def _prep(feat, hs):
    return pl.pallas_call(
        _prep_body,
        grid=(N // BLK,),
        in_specs=[
            pl.BlockSpec((BLK, 128), lambda i: (i, 0)),
            pl.BlockSpec((BLK, 128), lambda i: (i, 0)),
        ],
        out_specs=pl.BlockSpec((BLK, 128), lambda i: (i, 0)),
        out_shape=jax.ShapeDtypeStruct((N, 128), jnp.float32),
    )(feat, hs)


def _post1_body(a0_r, a1_r, hd_r, hs_r, W1_r, b1_r, y2_r):
    nd = _ns_from(hd_r[...])
    ns = _ns_from(hs_r[...])
    agg = (a0_r[...] + a1_r[...]) * nd
    h = _gelu(_dot(agg, W1_r[...]) + b1_r[...])
    y = h * ns
    y2_r[0] = y[:, :128]
    y2_r[1] = y[:, 128:]


def _post1(a0, a1, hd, hs, W1, b1):
    return pl.pallas_call(
        _post1_body,
        grid=(N // BLK,),
        in_specs=[
            pl.BlockSpec((BLK, 128), lambda i: (i, 0)),
            pl.BlockSpec((BLK, 128), lambda i: (i, 0)),
            pl.BlockSpec((BLK, 128), lambda i: (i, 0)),
            pl.BlockSpec((BLK, 128), lambda i: (i, 0)),
            pl.BlockSpec((128, HID), lambda i: (0, 0)),
            pl.BlockSpec((1, HID), lambda i: (0, 0)),
        ],
        out_specs=pl.BlockSpec((2, BLK, 128), lambda i: (0, i, 0)),
        out_shape=jax.ShapeDtypeStruct((2, N, 128), jnp.float32),
    )(a0, a1, hd, hs, W1, b1.reshape(1, HID))


def _post2_body(a0_r, a1_r, hd_r, hs_r, W2_r, b2_r, W3_r, t_r):
    nd = _ns_from(hd_r[...])
    ns = _ns_from(hs_r[...])
    W2 = W2_r[...]
    u = _gelu(_dot(a0_r[...] * nd, W2[:128, :]) +
              _dot(a1_r[...] * nd, W2[128:, :]) + b2_r[...])
    t_r[...] = _dot(u, W3_r[...]) * ns


def _post2(a0, a1, hd, hs, W2, b2, W3):
    return pl.pallas_call(
        _post2_body,
        grid=(N // BLK,),
        in_specs=[
            pl.BlockSpec((BLK, 128), lambda i: (i, 0)),
            pl.BlockSpec((BLK, 128), lambda i: (i, 0)),
            pl.BlockSpec((BLK, 128), lambda i: (i, 0)),
            pl.BlockSpec((BLK, 128), lambda i: (i, 0)),
            pl.BlockSpec((HID, HID), lambda i: (0, 0)),
            pl.BlockSpec((1, HID), lambda i: (0, 0)),
            pl.BlockSpec((HID, 128), lambda i: (0, 0)),
        ],
        out_specs=pl.BlockSpec((BLK, 128), lambda i: (i, 0)),
        out_shape=jax.ShapeDtypeStruct((N, 128), jnp.float32),
    )(a0, a1, hd, hs, W2, b2.reshape(1, HID), W3)


def _post3_body(a0_r, a1_r, hd_r, b3_r, lng_r, lnb_r, Wp1_r, bp1_r, Wp2_r,
                bp2_r, Wh1_r, bh1_r, wv_r, cb_r, h_r, z_r, sc_r):
    nd = _ns_from(hd_r[...])
    hh = _layernorm((a0_r[...] + a1_r[...]) * nd + b3_r[...],
                    lng_r[...], lnb_r[...])
    h_r[...] = hh
    zz = _dot(_gelu(_dot(hh, Wp1_r[...]) + bp1_r[...]), Wp2_r[...]) + bp2_r[...]
    nrm = jnp.sqrt(jnp.sum(zz * zz, axis=1, keepdims=True))
    z_r[...] = zz / jnp.maximum(nrm, 1e-12)
    A = _gelu(_dot(hh, Wh1_r[...]) + bh1_r[...])
    sc_r[...] = _dot(A, wv_r[...]) + cb_r[...]


def _post3(a0, a1, hd, b3, ln_g, ln_b, Wp1, bp1, Wp2, bp2, Wh1f, bh1f, wvp, cbv):
    row = pl.BlockSpec((BLK, 128), lambda i: (i, 0))
    c128 = pl.BlockSpec((1, 128), lambda i: (0, 0))
    return pl.pallas_call(
        _post3_body,
        grid=(N // BLK,),
        in_specs=[
            row, row,
            pl.BlockSpec((BLK, 128), lambda i: (i, 0)),
            c128, c128, c128,
            pl.BlockSpec((128, 128), lambda i: (0, 0)), c128,
            pl.BlockSpec((128, 128), lambda i: (0, 0)), c128,
            pl.BlockSpec((128, 512), lambda i: (0, 0)),
            pl.BlockSpec((1, 512), lambda i: (0, 0)),
            pl.BlockSpec((512, 128), lambda i: (0, 0)),
            c128,
        ],
        out_specs=[row, row, row],
        out_shape=[
            jax.ShapeDtypeStruct((N, 128), jnp.float32),
            jax.ShapeDtypeStruct((N, 128), jnp.float32),
            jax.ShapeDtypeStruct((N, 128), jnp.float32),
        ],
    )(a0, a1, hd, b3.reshape(1, 128), ln_g.reshape(1, 128),
      ln_b.reshape(1, 128), Wp1, bp1.reshape(1, 128), Wp2,
      bp2.reshape(1, 128), Wh1f, bh1f, wvp, cbv)


def _softmaxw_body(sc_r, h_r, w_r):
    s0 = sc_r[:, :1]
    m = jnp.max(s0)
    e = jnp.exp(s0 - m)
    w = e / jnp.sum(e)
    w_r[...] = w * h_r[...]


def _softmaxw(sc, h):
    return pl.pallas_call(
        _softmaxw_body,
        in_specs=[
            pl.BlockSpec((N, 128), lambda: (0, 0)),
            pl.BlockSpec((N, 128), lambda: (0, 0)),
        ],
        out_specs=pl.BlockSpec((N, 128), lambda: (0, 0)),
        out_shape=jax.ShapeDtypeStruct((N, 128), jnp.float32),
    )(sc, h)


def _contrast_kernel(z_blk_ref, z_all_ref, out_ref):
    i = pl.program_id(0)
    z_blk = z_blk_ref[...]
    s = lax.dot_general(
        z_blk, z_all_ref[...], (((1,), (1,)), ((), ())),
        preferred_element_type=jnp.float32,
        precision=_PREC,
    ) * (1.0 / TEMP)
    m = jnp.max(s, axis=1, keepdims=True)
    lse = jnp.log(jnp.sum(jnp.exp(s - m), axis=1, keepdims=True)) + m
    diag = jnp.sum(z_blk * z_blk, axis=1, keepdims=True) * (1.0 / TEMP)
    part = jnp.sum(lse - diag, axis=(0, 1), keepdims=True)

    @pl.when(i == 0)
    def _():
        out_ref[...] = jnp.zeros_like(out_ref)

    out_ref[...] += part


def _contrast(z):
    return pl.pallas_call(
        _contrast_kernel,
        grid=(N // ROW_BLK,),
        in_specs=[
            pl.BlockSpec((ROW_BLK, OUT), lambda i: (i, 0)),
            pl.BlockSpec((N, OUT), lambda i: (0, 0)),
        ],
        out_specs=pl.BlockSpec((1, 1), lambda i: (0, 0)),
        out_shape=jax.ShapeDtypeStruct((1, 1), jnp.float32),
    )(z, z)


def _final_body(b0_r, b1_r, Wc1_r, bc1_r, lg_r, lb_r, Wc2_r, bc2_r, oh_r,
                tot_r, logits_r, loss_r):
    bf = b0_r[...] + b1_r[...]
    x = _gelu(_layernorm(_dot(bf, Wc1_r[...]) + bc1_r[...], lg_r[...], lb_r[...]))
    logits = _dot(x, Wc2_r[...]) + bc2_r[...]
    logits_r[...] = logits
    m = jnp.max(logits, axis=1, keepdims=True)
    lse = jnp.log(jnp.sum(jnp.exp(logits - m), axis=1, keepdims=True)) + m
    lp = logits - lse
    nll = -jnp.sum(lp * oh_r[...], axis=1, keepdims=True)
    pmean = -jnp.mean(lp, axis=1, keepdims=True)
    cls = jnp.sum(0.9 * nll + 0.1 * pmean, axis=(0, 1), keepdims=True) / NB
    loss_r[...] = tot_r[...] * (0.6 / N) + cls * 0.4


def _final(b0, b1, Wc1, bc1, lnc_g, lnc_b, Wc2, bc2, oh, total):
    full = lambda s: pl.BlockSpec(s, lambda: tuple(0 for _ in s))
    return pl.pallas_call(
        _final_body,
        in_specs=[
            full((NB, 128)), full((NB, 128)),
            full((128, 128)), full((1, 128)), full((1, 128)), full((1, 128)),
            full((128, NC)), full((1, NC)), full((NB, NC)), full((1, 1)),
        ],
        out_specs=[full((NB, NC)), full((1, 1))],
        out_shape=[
            jax.ShapeDtypeStruct((NB, NC), jnp.float32),
            jax.ShapeDtypeStruct((1, 1), jnp.float32),
        ],
    )(b0, b1, Wc1, bc1.reshape(1, 128), lnc_g.reshape(1, 128),
      lnc_b.reshape(1, 128), Wc2, bc2.reshape(1, NC), oh, total)


def kernel(feat, edge_index, bag_indices, labels, W1, b1, W2, b2, W3, b3,
           ln_g, ln_b, Wh1, bh1, Wh2, bh2, Wp1, bp1, Wp2, bp2, Wc1, bc1,
           lnc_g, lnc_b, Wc2, bc2):
    PW = 2560
    srcw2 = jnp.pad(edge_index[0], (0, PW * 128 - E)).reshape(PW, 128)
    dstw2 = jnp.pad(edge_index[1], (0, PW * 128 - E)).reshape(PW, 128)
    zN = jnp.zeros((N, 128), jnp.float32)
    zB = jnp.zeros((NB, 128), jnp.float32)
    o128 = jnp.ones((128, 128), jnp.float32)

    hist = _sc_hist(jnp.concatenate([srcw2, dstw2]), zN, o128, EW)
    hs, hd = hist[0], hist[1]

    y0 = _prep(feat, hs)
    agg1 = _sc_segsum_split(y0, srcw2, dstw2, zN, EW)
    y2 = _post1(agg1[0], agg1[1], hd, hs, W1, b1)
    src2w = jnp.concatenate([srcw2, srcw2 + N])
    agg2 = _sc_segsum_feat(y2.reshape(2 * N, 128), src2w, dstw2, zN, EW)
    t = _post2(agg2[0], agg2[1], hd, hs, W2, b2, W3)
    agg3 = _sc_segsum_split(t, srcw2, dstw2, zN, EW)

    Wh1f = Wh1.transpose(1, 0, 2).reshape(128, 512)
    bh1f = bh1.reshape(1, 512)
    wvp = jnp.pad(Wh2.reshape(512, 1) / 4.0, ((0, 0), (0, 127)))
    cbv = jnp.full((1, 128), jnp.mean(bh2), jnp.float32)
    h, z, sc = _post3(agg3[0], agg3[1], hd, b3, ln_g, ln_b, Wp1, bp1, Wp2,
                      bp2, Wh1f, bh1f, wvp, cbv)

    total = _contrast(z)
    weighted = _softmaxw(sc, h)

    PWB = 64
    bagw2 = jnp.pad(bag_indices.reshape(NB * BS),
                    (0, PWB * 128 - NB * BS)).reshape(PWB, 128)
    bagidw2 = jnp.pad(jnp.repeat(jnp.arange(NB, dtype=jnp.int32), BS),
                      (0, PWB * 128 - NB * BS)).reshape(PWB, 128)
    bagf = _sc_segsum_split(weighted, bagw2, bagidw2, zB, NB * BS // 128)

    oh = (labels[:, None] == jnp.arange(NC, dtype=labels.dtype)[None, :]
          ).astype(jnp.float32)
    logits, loss = _final(bagf[0], bagf[1], Wc1, bc1, lnc_g, lnc_b, Wc2, bc2,
                          oh, total)
    return logits, loss[0, 0]
